# baseline (speedup 1.0000x reference)
import functools

import jax
import jax.numpy as jnp
from jax import lax
from jax.experimental import pallas as pl
from jax.experimental.pallas import tpu as pltpu
from jax.experimental.pallas import tpu_sc as plsc

_D = 1024
_NL = 3
_B = 16
_L = 512
_RW = 16


def _sc_body(src_t, tgt_t, o0, o1, o2, o3, o4, o5,
             buf0, buf1, zbuf, zidx,
             ix00, ix01, ix02, ix10, ix11, ix12,
             gsem0, gsem1, ssem0, ssem1):
    cid = lax.axis_index("c")
    sid = lax.axis_index("s")
    wid = sid * 2 + cid
    rb = jnp.minimum(wid * _RW, 511 - _RW)

    sides = (
        (src_t, (o0, o1, o2)),
        (tgt_t, (o3, o4, o5)),
    )
    bufs = (buf0, buf1)
    gsems = (gsem0, gsem1)
    ssems = (ssem0, ssem1)
    idxs = ((ix00, ix01, ix02), (ix10, ix11, ix12))
    iota16 = lax.iota(jnp.int32, 16)
    zeros16 = jnp.zeros((16,), jnp.float32)

    @pl.when(wid == 0)
    def _():
        for r in range(16):
            for t in range(_D // 16):
                zbuf[r, pl.ds(t * 16, 16)] = zeros16
        zidx[pl.ds(0, 16)] = iota16 * _L
        for _, outs in sides:
            for out2d in outs:
                pltpu.async_copy(zbuf, out2d.at[zidx], ssems[0]).wait()

    chunks = []
    for e_t, outs in sides:
        for g in range(_RW):
            chunks.append((e_t, outs, g))
    n = len(chunks)

    def gather(i):
        e_t, _, g = chunks[i]
        p = i % 2
        return pltpu.async_copy(
            e_t.at[pl.ds(rb + g, 1), :, :], bufs[p], gsems[p])

    def scatter(i):
        _, outs, g = chunks[i]
        p = i % 2
        hs = []
        for l, out2d in enumerate(outs):
            ix = idxs[p][l]
            ix[pl.ds(0, 16)] = iota16 * _L + (rb + g + 1)
            hs.append(pltpu.async_copy(
                bufs[p].at[0, :, pl.ds(l * _D, _D)], out2d.at[ix], ssems[p]))
        return hs

    g = [None, None]
    s = [None, None]
    g[0] = gather(0)
    for i in range(n):
        p = i % 2
        q = (i + 1) % 2
        if i + 1 < n:
            if s[q] is not None:
                for h in s[q]:
                    h.wait()
                s[q] = None
            g[q] = gather(i + 1)
        g[p].wait()
        s[p] = scatter(i)
    for s_ in s:
        if s_ is not None:
            for h in s_:
                h.wait()


def kernel(elmo_src, elmo_tgt):
    mesh = plsc.VectorSubcoreMesh(core_axis_name="c", subcore_axis_name="s")
    out_struct = jax.ShapeDtypeStruct((_B * _L, _D), jnp.float32)
    kern = functools.partial(
        pl.kernel,
        out_type=[out_struct] * 6,
        mesh=mesh,
        scratch_types=[
            pltpu.VMEM((1, _B, _NL * _D), jnp.float32),
            pltpu.VMEM((1, _B, _NL * _D), jnp.float32),
            pltpu.VMEM((16, _D), jnp.float32),
            pltpu.VMEM((16,), jnp.int32),
            pltpu.VMEM((16,), jnp.int32),
            pltpu.VMEM((16,), jnp.int32),
            pltpu.VMEM((16,), jnp.int32),
            pltpu.VMEM((16,), jnp.int32),
            pltpu.VMEM((16,), jnp.int32),
            pltpu.VMEM((16,), jnp.int32),
            pltpu.SemaphoreType.DMA,
            pltpu.SemaphoreType.DMA,
            pltpu.SemaphoreType.DMA,
            pltpu.SemaphoreType.DMA,
        ],
    )(_sc_body)
    src_t = jnp.transpose(elmo_src, (1, 0, 2))
    tgt_t = jnp.transpose(elmo_tgt, (1, 0, 2))
    outs = kern(src_t, tgt_t)
    return tuple(o.reshape(_B, _L, _D) for o in outs)
